# baseline (speedup 1.0000x reference)
import functools

import numpy as np
import jax
import jax.numpy as jnp
from jax import lax
from jax.experimental import pallas as pl
from jax.experimental.pallas import tpu as pltpu
from jax.experimental.pallas import tpu_sc as plsc

NUM_FIELDS = 26
EMBED_DIM = 16
BATCH = 16384
ROWS = BATCH * NUM_FIELDS
FIELD_SIZE = 100000
_OFFSETS_NP = (np.arange(NUM_FIELDS) * FIELD_SIZE).astype(np.int32)
NUM_PAIRS = NUM_FIELDS * (NUM_FIELDS - 1) // 2

_iu, _ju = np.triu_indices(NUM_FIELDS, k=1)
_ref_pos = {(int(i), int(j)): p for p, (i, j) in enumerate(zip(_iu, _ju))}
_PERM = np.array(
    [_ref_pos[(i, i + k)] for k in range(1, NUM_FIELDS) for i in range(NUM_FIELDS - k)],
    dtype=np.int32,
)

_NC, _NS = 2, 16
_NW = _NC * _NS
_ROWS_PER_W = ROWS // _NW
_CHUNK = 3328
_NCHUNK = _ROWS_PER_W // _CHUNK

_sc_mesh = plsc.VectorSubcoreMesh(core_axis_name="c", subcore_axis_name="s")


@functools.partial(
    pl.kernel,
    mesh=_sc_mesh,
    out_type=jax.ShapeDtypeStruct((ROWS, EMBED_DIM), jnp.float32),
    scratch_types=[
        pltpu.VMEM((_CHUNK,), jnp.int32),
        pltpu.VMEM((_CHUNK, EMBED_DIM), jnp.float32),
        pltpu.SemaphoreType.DMA,
    ],
    compiler_params=pltpu.CompilerParams(use_tc_tiling_on_sc=False),
)
def _sc_gather(idx_hbm, table_hbm, out_hbm, idx_v, rows_v, sem):
    wid = lax.axis_index("s") * _NC + lax.axis_index("c")
    base = wid * _ROWS_PER_W
    for c in range(_NCHUNK):
        off = base + c * _CHUNK
        pltpu.sync_copy(idx_hbm.at[pl.ds(off, _CHUNK)], idx_v)
        pltpu.async_copy(table_hbm.at[idx_v], rows_v, sem).wait()
        pltpu.sync_copy(rows_v, out_hbm.at[pl.ds(off, _CHUNK)])


_BB = 512
_GRID = BATCH // _BB
_EW = NUM_FIELDS * EMBED_DIM


def _tc_body(emb_ref, w1t_ref, b1_ref, w2t_ref, b2_ref, wot_ref, bo_ref, out_ref):
    e = emb_ref[...]
    et = e.T
    parts = []
    for k in range(1, NUM_FIELDS):
        n = NUM_FIELDS - k
        a = et[: n * EMBED_DIM, :]
        b = et[k * EMBED_DIM :, :]
        prod = (a * b).reshape(n, EMBED_DIM, _BB)
        parts.append(jnp.sum(prod, axis=1))
    hT = jnp.concatenate(parts, axis=0)
    z1 = jnp.dot(w1t_ref[...], hT, preferred_element_type=jnp.float32)
    h1 = jnp.maximum(z1 + b1_ref[...], 0.0)
    z2 = jnp.dot(w2t_ref[...], h1, preferred_element_type=jnp.float32)
    h2 = jnp.maximum(z2 + b2_ref[...], 0.0)
    o = jnp.dot(wot_ref[...], h2, preferred_element_type=jnp.float32) + bo_ref[...]
    out_ref[...] = jax.nn.sigmoid(o)


_tc_call = pl.pallas_call(
    _tc_body,
    grid=(_GRID,),
    in_specs=[
        pl.BlockSpec((_BB, _EW), lambda i: (i, 0)),
        pl.BlockSpec((512, NUM_PAIRS), lambda i: (0, 0)),
        pl.BlockSpec((512, 1), lambda i: (0, 0)),
        pl.BlockSpec((256, 512), lambda i: (0, 0)),
        pl.BlockSpec((256, 1), lambda i: (0, 0)),
        pl.BlockSpec((1, 256), lambda i: (0, 0)),
        pl.BlockSpec((1, 1), lambda i: (0, 0)),
    ],
    out_specs=pl.BlockSpec((1, _BB), lambda i: (0, i)),
    out_shape=jax.ShapeDtypeStruct((1, BATCH), jnp.float32),
)


def kernel(x, table, W1, b1, W2, b2, Wout, bout):
    offsets = jnp.asarray(_OFFSETS_NP)
    idx_flat = (x + offsets[None, :]).reshape(ROWS)
    emb = _sc_gather(idx_flat, table)
    emb2 = emb.reshape(BATCH, _EW)
    w1t = W1[jnp.asarray(_PERM), :].T
    out = _tc_call(
        emb2,
        w1t,
        b1.reshape(512, 1),
        W2.T,
        b2.reshape(256, 1),
        Wout.T,
        bout.reshape(1, 1),
    )
    return out.reshape(BATCH, 1)
